# baseline (speedup 1.0000x reference)
import functools

import jax
import jax.numpy as jnp
from jax import lax
from jax.experimental import pallas as pl
from jax.experimental.pallas import tpu as pltpu
from jax.experimental.pallas import tpu_sc as plsc

_TOP_K = 8
_L = 16


def _logits_block(x_ref, wt_ref, b_ref, out_ref):
    out_ref[...] = jnp.dot(
        x_ref[...], wt_ref[...], preferred_element_type=jnp.float32
    ) + b_ref[...]


def _merge16(ka, va, kb, vb, mask8):
    kc = jnp.where(mask8, ka, lax.rev(kb, (0,)))
    vc = jnp.where(mask8, va, lax.rev(vb, (0,)))
    return plsc.sort_key_val(kc, vc, descending=True)


def _sc_topk(logits_hbm, gates_hbm, idx_hbm, lg_v, g_v, i_v, *, tpw, ne):
    wid = lax.axis_index("s") * 2 + lax.axis_index("c")
    base = wid * tpw
    pltpu.sync_copy(logits_hbm.at[pl.ds(base * ne, tpw * ne)], lg_v)
    iota = lax.iota(jnp.int32, _L)
    mask8 = iota < _TOP_K

    def body(t, carry):
        off = t * ne
        ks, vs = [], []
        for j in range(ne // _L):
            kj, vj = plsc.sort_key_val(
                lg_v[pl.ds(off + j * _L, _L)], iota + j * _L,
                descending=True)
            ks.append(kj)
            vs.append(vj)
        k01, v01 = _merge16(ks[0], vs[0], ks[1], vs[1], mask8)
        k23, v23 = _merge16(ks[2], vs[2], ks[3], vs[3], mask8)
        k, v = _merge16(k01, v01, k23, v23, mask8)
        mx = jnp.max(k)
        e = jnp.where(mask8, jnp.exp(k - mx), 0.0)
        g = e / jnp.sum(e)
        plsc.store_compressed(g_v.at[pl.ds(t * _TOP_K, _L)], g, mask=mask8)
        plsc.store_compressed(i_v.at[pl.ds(t * _TOP_K, _L)], v, mask=mask8)
        return carry

    lax.fori_loop(0, tpw, body, 0)
    pltpu.sync_copy(g_v.at[pl.ds(0, tpw * _TOP_K)],
                    gates_hbm.at[pl.ds(base * _TOP_K, tpw * _TOP_K)])
    pltpu.sync_copy(i_v.at[pl.ds(0, tpw * _TOP_K)],
                    idx_hbm.at[pl.ds(base * _TOP_K, tpw * _TOP_K)])


@jax.jit
def kernel(x, W, b):
    B, S, D = x.shape
    E = W.shape[0]
    T = B * S
    x2 = x.reshape(T, D)
    bt = 512
    while T % bt:
        bt //= 2
    logits = pl.pallas_call(
        _logits_block,
        grid=(T // bt,),
        in_specs=[
            pl.BlockSpec((bt, D), lambda i: (i, 0)),
            pl.BlockSpec((D, E), lambda i: (0, 0)),
            pl.BlockSpec((1, E), lambda i: (0, 0)),
        ],
        out_specs=pl.BlockSpec((bt, E), lambda i: (i, 0)),
        out_shape=jax.ShapeDtypeStruct((T, E), jnp.float32),
    )(x2, W.T, b.reshape(1, E))

    nw = 32
    tpw = T // nw
    mesh = plsc.VectorSubcoreMesh(core_axis_name="c", subcore_axis_name="s")
    sc = pl.kernel(
        functools.partial(_sc_topk, tpw=tpw, ne=E),
        out_type=[
            jax.ShapeDtypeStruct((T * _TOP_K,), jnp.float32),
            jax.ShapeDtypeStruct((T * _TOP_K,), jnp.int32),
        ],
        mesh=mesh,
        compiler_params=pltpu.CompilerParams(needs_layout_passes=False),
        scratch_types=[
            pltpu.VMEM((tpw * E,), jnp.float32),
            pltpu.VMEM((tpw * _TOP_K + _L,), jnp.float32),
            pltpu.VMEM((tpw * _TOP_K + _L,), jnp.int32),
        ],
    )
    gates_f, idx_f = sc(logits.reshape(T * E))
    return (gates_f.reshape(B, S, _TOP_K), idx_f.reshape(B, S, _TOP_K))


# speedup vs baseline: 1.2101x; 1.2101x over previous
import functools

import jax
import jax.numpy as jnp
from jax import lax
from jax.experimental import pallas as pl
from jax.experimental.pallas import tpu as pltpu
from jax.experimental.pallas import tpu_sc as plsc

_TOP_K = 8
_L = 16


def _logits_block(x_ref, wt_ref, b_ref, out_ref):
    out_ref[...] = jnp.dot(
        x_ref[...], wt_ref[...], preferred_element_type=jnp.float32
    ) + b_ref[...]


def _merge16(ka, va, kb, vb, mask8):
    kc = jnp.where(mask8, ka, lax.rev(kb, (0,)))
    vc = jnp.where(mask8, va, lax.rev(vb, (0,)))
    return plsc.sort_key_val(kc, vc, descending=True)


def _sc_topk(logits_hbm, gates_hbm, idx_hbm, lg_v, g_v, i_v, *, tpw, ne):
    wid = lax.axis_index("s") * 2 + lax.axis_index("c")
    base = wid * tpw
    pltpu.sync_copy(logits_hbm.at[pl.ds(base * ne, tpw * ne)], lg_v)
    iota = lax.iota(jnp.int32, _L)
    mask8 = iota < _TOP_K

    @plsc.parallel_loop(0, tpw, 1, unroll=4)
    def body(t):
        off = t * ne
        ks, vs = [], []
        for j in range(ne // _L):
            kj, vj = plsc.sort_key_val(
                lg_v[pl.ds(off + j * _L, _L)], iota + j * _L,
                descending=True)
            ks.append(kj)
            vs.append(vj)
        k01, v01 = _merge16(ks[0], vs[0], ks[1], vs[1], mask8)
        k23, v23 = _merge16(ks[2], vs[2], ks[3], vs[3], mask8)
        k, v = _merge16(k01, v01, k23, v23, mask8)
        mx = jnp.max(k)
        e = jnp.where(mask8, jnp.exp(k - mx), 0.0)
        g = e / jnp.sum(e)
        plsc.store_compressed(g_v.at[pl.ds(t * _TOP_K, _L)], g, mask=mask8)
        plsc.store_compressed(i_v.at[pl.ds(t * _TOP_K, _L)], v, mask=mask8)

    pltpu.sync_copy(g_v.at[pl.ds(0, tpw * _TOP_K)],
                    gates_hbm.at[pl.ds(base * _TOP_K, tpw * _TOP_K)])
    pltpu.sync_copy(i_v.at[pl.ds(0, tpw * _TOP_K)],
                    idx_hbm.at[pl.ds(base * _TOP_K, tpw * _TOP_K)])


@jax.jit
def kernel(x, W, b):
    B, S, D = x.shape
    E = W.shape[0]
    T = B * S
    x2 = x.reshape(T, D)
    bt = 512
    while T % bt:
        bt //= 2
    logits = pl.pallas_call(
        _logits_block,
        grid=(T // bt,),
        in_specs=[
            pl.BlockSpec((bt, D), lambda i: (i, 0)),
            pl.BlockSpec((D, E), lambda i: (0, 0)),
            pl.BlockSpec((1, E), lambda i: (0, 0)),
        ],
        out_specs=pl.BlockSpec((bt, E), lambda i: (i, 0)),
        out_shape=jax.ShapeDtypeStruct((T, E), jnp.float32),
    )(x2, W.T, b.reshape(1, E))

    nw = 32
    tpw = T // nw
    mesh = plsc.VectorSubcoreMesh(core_axis_name="c", subcore_axis_name="s")
    sc = pl.kernel(
        functools.partial(_sc_topk, tpw=tpw, ne=E),
        out_type=[
            jax.ShapeDtypeStruct((T * _TOP_K,), jnp.float32),
            jax.ShapeDtypeStruct((T * _TOP_K,), jnp.int32),
        ],
        mesh=mesh,
        compiler_params=pltpu.CompilerParams(needs_layout_passes=False),
        scratch_types=[
            pltpu.VMEM((tpw * E,), jnp.float32),
            pltpu.VMEM((tpw * _TOP_K + _L,), jnp.float32),
            pltpu.VMEM((tpw * _TOP_K + _L,), jnp.int32),
        ],
    )
    gates_f, idx_f = sc(logits.reshape(T * E))
    return (gates_f.reshape(B, S, _TOP_K), idx_f.reshape(B, S, _TOP_K))
